# Initial kernel scaffold; baseline (speedup 1.0000x reference)
import jax
import jax.numpy as jnp
from jax.experimental import pallas as pl


def kernel(x, edge_index, edge_attr, W_proj, b_proj, W_e1, b_e1, W_e2, b_e2, b_conv, W_ih, W_hh, b_ih, b_hh, Wl_ih, Wl_hh, bl_ih, bl_hh, W_m1, b_m1, W_m2, b_m2):
    raise NotImplementedError("write your pallas kernel here")


# speedup vs baseline: 1.2265x; 1.2265x over previous
import functools

import jax
import jax.numpy as jnp
from jax import lax
from jax.experimental import pallas as pl
from jax.experimental.pallas import tpu as pltpu
from jax.experimental.pallas import tpu_sc as plsc

_NC, _NS = 2, 16
_NW = _NC * _NS
_CHUNK = 125


def _hi(a, b):
    return jnp.dot(a, b, precision=lax.Precision.HIGHEST,
                   preferred_element_type=jnp.float32)


def _proj_body(x_ref, w_ref, b_ref, o_ref):
    o_ref[...] = jnp.maximum(_hi(x_ref[...], w_ref[...]) + b_ref[...], 0.0)


def _edge_f_body(ea_ref, w_ref, b_ref, o_ref):
    o_ref[...] = jnp.maximum(_hi(ea_ref[...], w_ref[...]) + b_ref[...], 0.0)


def _msg_body(hs_ref, f_ref, w2t_ref, be2_ref, r_ref, s_ref, o_ref):
    ew = _hi(f_ref[...], w2t_ref[...]) + be2_ref[...]
    he = _hi(hs_ref[...], r_ref[...])
    o_ref[...] = _hi(he * ew, s_ref[...])


def _gru_body(n, parts_ref, h_ref, wih_ref, whh_ref, bih_ref, bhh_ref,
              bconv_ref, o_ref):
    agg = parts_ref[0, :n, :] + parts_ref[1, :n, :] + bconv_ref[...]
    nf = jnp.maximum(agg, 0.0)
    gi = _hi(nf, wih_ref[...]) + bih_ref[...]
    gh = _hi(h_ref[...], whh_ref[...]) + bhh_ref[...]
    i_r, i_z, i_n = jnp.split(gi, 3, axis=-1)
    h_r, h_z, h_n = jnp.split(gh, 3, axis=-1)
    r = jax.nn.sigmoid(i_r + h_r)
    z = jax.nn.sigmoid(i_z + h_z)
    nn = jnp.tanh(i_n + r * h_n)
    o_ref[...] = (1.0 - z) * nn + z * h_ref[...]


def _s2s_body(steps, h_ref, wlih_ref, wlhh_ref, blih_ref, blhh_ref,
              wm1_ref, bm1_ref, wm2_ref, bm2_ref, o_ref):
    h = h_ref[...]
    hdim = h.shape[1]
    q_star = jnp.zeros((1, 2 * hdim), jnp.float32)
    lh = jnp.zeros((1, hdim), jnp.float32)
    lc = jnp.zeros((1, hdim), jnp.float32)
    for _ in range(steps):
        gates = (_hi(q_star, wlih_ref[...]) + blih_ref[...]
                 + _hi(lh, wlhh_ref[...]) + blhh_ref[...])
        ii, ff, gg, oo = jnp.split(gates, 4, axis=-1)
        lc = jax.nn.sigmoid(ff) * lc + jax.nn.sigmoid(ii) * jnp.tanh(gg)
        lh = jax.nn.sigmoid(oo) * jnp.tanh(lc)
        q = lh
        e = jnp.sum(h * q, axis=-1, keepdims=True)
        e = e - jnp.max(e)
        a = jnp.exp(e)
        alpha = a / jnp.sum(a)
        readout = jnp.sum(alpha * h, axis=0, keepdims=True)
        q_star = jnp.concatenate([q, readout], axis=-1)
    o_ref[...] = _hi(jnp.maximum(_hi(q_star, wm1_ref[...]) + bm1_ref[...],
                                 0.0), wm2_ref[...]) + bm2_ref[...]


def _make_sc_gather(n, h, e):
    ep = e // _NW
    ch = ep // _CHUNK
    mesh = plsc.VectorSubcoreMesh(core_axis_name="c", subcore_axis_name="s",
                                  num_cores=_NC, num_subcores=_NS)

    @functools.partial(
        pl.kernel,
        out_type=jax.ShapeDtypeStruct((e, h), jnp.float32),
        mesh=mesh,
        compiler_params=pltpu.CompilerParams(use_tc_tiling_on_sc=False),
        scratch_types=[
            pltpu.VMEM((ch, _CHUNK), jnp.int32),
            pltpu.VMEM((ep, h), jnp.float32),
            pltpu.SemaphoreType.DMA,
        ],
    )
    def sc_gather(idx_hbm, tab_hbm, out_hbm, idx_v, rows_v, sem):
        c = lax.axis_index("c")
        s = lax.axis_index("s")
        wid = s * _NC + c
        pltpu.sync_copy(idx_hbm.at[wid], idx_v)
        descs = [
            pltpu.async_copy(tab_hbm.at[idx_v.at[j]],
                             rows_v.at[pl.ds(j * _CHUNK, _CHUNK), :], sem)
            for j in range(ch)
        ]
        for d in descs:
            d.wait()
        pltpu.sync_copy(rows_v, out_hbm.at[pl.ds(wid * ep, ep), :])

    return sc_gather


def _make_sc_scatter(n_pad, h, e):
    ep = e // _NW
    ch = ep // _CHUNK
    rows = n_pad // _NS
    mesh = plsc.VectorSubcoreMesh(core_axis_name="c", subcore_axis_name="s",
                                  num_cores=_NC, num_subcores=_NS)

    @functools.partial(
        pl.kernel,
        out_type=jax.ShapeDtypeStruct((_NC, n_pad, h), jnp.float32),
        mesh=mesh,
        compiler_params=pltpu.CompilerParams(use_tc_tiling_on_sc=False),
        scratch_types=[
            pltpu.VMEM((ch, _CHUNK), jnp.int32),
            pltpu.VMEM((ep, h), jnp.float32),
            pltpu.VMEM((rows, h), jnp.float32),
            pltpu.VMEM_SHARED((n_pad, h), jnp.float32),
            pltpu.SemaphoreType.DMA,
        ],
    )
    def sc_scatter(idx_hbm, m_hbm, z_hbm, out_hbm, idx_v, m_v, z_v, acc_sh,
                   sem):
        c = lax.axis_index("c")
        s = lax.axis_index("s")
        wid = s * _NC + c
        pltpu.sync_copy(z_hbm, z_v)
        pltpu.sync_copy(z_v, acc_sh.at[pl.ds(s * rows, rows), :])
        pltpu.sync_copy(idx_hbm.at[wid], idx_v)
        pltpu.sync_copy(m_hbm.at[pl.ds(wid * ep, ep), :], m_v)
        plsc.subcore_barrier()
        for j in range(ch):
            pltpu.sync_copy(m_v.at[pl.ds(j * _CHUNK, _CHUNK), :],
                            acc_sh.at[idx_v.at[j]], add=True)
        plsc.subcore_barrier()
        pltpu.sync_copy(acc_sh.at[pl.ds(s * rows, rows), :],
                        out_hbm.at[c, pl.ds(s * rows, rows), :])

    return sc_scatter


_STEPS = 3
_S2S_STEPS = 6


def kernel(x, edge_index, edge_attr, W_proj, b_proj, W_e1, b_e1, W_e2, b_e2,
           b_conv, W_ih, W_hh, b_ih, b_hh, Wl_ih, Wl_hh, bl_ih, bl_hh,
           W_m1, b_m1, W_m2, b_m2):
    n, d_in = x.shape
    e, d_e = edge_attr.shape
    hd = W_proj.shape[0]
    eh = W_e1.shape[0]
    n_pad = ((n + _NS - 1) // _NS) * _NS
    ep = e // _NW
    ch = ep // _CHUNK
    assert ep * _NW == e and ch * _CHUNK == ep

    src3 = edge_index[0].reshape(_NW, ch, _CHUNK)
    dst3 = edge_index[1].reshape(_NW, ch, _CHUNK)

    ar_h = jnp.arange(hd)
    ar_hh = jnp.arange(hd * hd)
    r_mat = (ar_hh[None, :] // hd == ar_h[:, None]).astype(jnp.float32)
    s_mat = (ar_hh[:, None] % hd == ar_h[None, :]).astype(jnp.float32)
    zeros_rows = jnp.zeros((n_pad // _NS, hd), jnp.float32)

    h0 = pl.pallas_call(
        _proj_body,
        out_shape=jax.ShapeDtypeStruct((n, hd), jnp.float32),
    )(x, W_proj.T, b_proj.reshape(1, hd))

    be = 8000
    f = pl.pallas_call(
        _edge_f_body,
        grid=(e // be,),
        in_specs=[
            pl.BlockSpec((be, d_e), lambda i: (i, 0)),
            pl.BlockSpec((d_e, eh), lambda i: (0, 0)),
            pl.BlockSpec((1, eh), lambda i: (0, 0)),
        ],
        out_specs=pl.BlockSpec((be, eh), lambda i: (i, 0)),
        out_shape=jax.ShapeDtypeStruct((e, eh), jnp.float32),
    )(edge_attr, W_e1.T, b_e1.reshape(1, eh))

    sc_gather = _make_sc_gather(n, hd, e)
    sc_scatter = _make_sc_scatter(n_pad, hd, e)

    bm = 2000
    msg_call = pl.pallas_call(
        _msg_body,
        grid=(e // bm,),
        in_specs=[
            pl.BlockSpec((bm, hd), lambda i: (i, 0)),
            pl.BlockSpec((bm, eh), lambda i: (i, 0)),
            pl.BlockSpec((eh, hd * hd), lambda i: (0, 0)),
            pl.BlockSpec((1, hd * hd), lambda i: (0, 0)),
            pl.BlockSpec((hd, hd * hd), lambda i: (0, 0)),
            pl.BlockSpec((hd * hd, hd), lambda i: (0, 0)),
        ],
        out_specs=pl.BlockSpec((bm, hd), lambda i: (i, 0)),
        out_shape=jax.ShapeDtypeStruct((e, hd), jnp.float32),
    )

    gru_call = pl.pallas_call(
        functools.partial(_gru_body, n),
        out_shape=jax.ShapeDtypeStruct((n, hd), jnp.float32),
    )

    w2t = W_e2.T
    be2 = b_e2.reshape(1, hd * hd)
    wih_t = W_ih.T
    whh_t = W_hh.T
    bih = b_ih.reshape(1, 3 * hd)
    bhh = b_hh.reshape(1, 3 * hd)
    bconv = b_conv.reshape(1, hd)

    h = h0
    for _ in range(_STEPS):
        h_src = sc_gather(src3, h)
        m = msg_call(h_src, f, w2t, be2, r_mat, s_mat)
        parts = sc_scatter(dst3, m, zeros_rows)
        h = gru_call(parts, h, wih_t, whh_t, bih, bhh, bconv)

    out = pl.pallas_call(
        functools.partial(_s2s_body, _S2S_STEPS),
        out_shape=jax.ShapeDtypeStruct((1, W_m2.shape[0]), jnp.float32),
    )(h, Wl_ih.T, Wl_hh.T, bl_ih.reshape(1, 4 * hd), bl_hh.reshape(1, 4 * hd),
      W_m1.T, b_m1.reshape(1, hd), W_m2.T, b_m2.reshape(1, W_m2.shape[0]))
    return out


# speedup vs baseline: 2.8433x; 2.3181x over previous
import functools

import jax
import jax.numpy as jnp
from jax import lax
from jax.experimental import pallas as pl
from jax.experimental.pallas import tpu as pltpu
from jax.experimental.pallas import tpu_sc as plsc

_NC, _NS = 2, 16
_NW = _NC * _NS
_CHUNK = 125
_G = 8


def _hi(a, b):
    return jnp.dot(a, b, precision=lax.Precision.HIGHEST,
                   preferred_element_type=jnp.float32)


def _bf(a, b):
    return jnp.dot(a, b, preferred_element_type=jnp.float32)


def _split3(a):
    hi = a.astype(jnp.bfloat16)
    r1 = a - hi.astype(jnp.float32)
    lo = r1.astype(jnp.bfloat16)
    lo2 = (r1 - lo.astype(jnp.float32)).astype(jnp.bfloat16)
    return hi, lo, lo2


def _onehot_exact(a, p_ref):
    hi, lo, lo2 = _split3(a)
    return _bf(hi, p_ref[...]) + _bf(lo, p_ref[...]) + _bf(lo2, p_ref[...])


def _proj_body(x_ref, w_ref, b_ref, o_ref):
    o_ref[...] = jnp.maximum(_bf(x_ref[...], w_ref[...]) + b_ref[...], 0.0)


def _edge_f_body(ea_ref, w_ref, b_ref, o_ref):
    o_ref[...] = jnp.maximum(_bf(ea_ref[...], w_ref[...]) + b_ref[...], 0.0)


def _msg_body(hs_ref, f_ref, w2_ref, be2_ref, r_ref, s_ref, o_ref):
    ew = _bf(f_ref[...], w2_ref[...]) + be2_ref[...]
    he = _onehot_exact(hs_ref[...], r_ref)
    o_ref[...] = _onehot_exact(he * ew, s_ref)


def _gru_body(hrows, parts_ref, h_ref, wr_ref, wz_ref, wn_ref, ur_ref,
              uz_ref, un_ref, bi_ref, bh_ref, bconv_ref, o_ref):
    agg = parts_ref[:hrows, :] + parts_ref[hrows:2 * hrows, :] + bconv_ref[...]
    nf = jnp.maximum(agg, 0.0)
    h = h_ref[...]
    ir = _bf(nf, wr_ref[...]) + bi_ref[0:1, :]
    iz = _bf(nf, wz_ref[...]) + bi_ref[1:2, :]
    inn = _bf(nf, wn_ref[...]) + bi_ref[2:3, :]
    hr = _bf(h, ur_ref[...]) + bh_ref[0:1, :]
    hz = _bf(h, uz_ref[...]) + bh_ref[1:2, :]
    hn = _bf(h, un_ref[...]) + bh_ref[2:3, :]
    r = jax.nn.sigmoid(ir + hr)
    z = jax.nn.sigmoid(iz + hz)
    nn = jnp.tanh(inn + r * hn)
    o_ref[...] = (1.0 - z) * nn + z * h


def _s2s_body(steps, hd, h_ref, efold_ref, aexp_ref, rfold_ref, wlih_ref,
              wlhh_ref, blih_ref, blhh_ref, wm1_ref, bm1_ref, wm2_ref,
              bm2_ref, o_ref):
    h = h_ref[...]
    q_star = jnp.zeros((1, 2 * hd), jnp.float32)
    lh = jnp.zeros((1, hd), jnp.float32)
    lc = jnp.zeros((1, hd), jnp.float32)
    for _ in range(steps):
        gates = (_bf(q_star, wlih_ref[...]) + blih_ref[...]
                 + _bf(lh, wlhh_ref[...]) + blhh_ref[...])
        ii, ff, gg, oo = jnp.split(gates, 4, axis=-1)
        lc = jax.nn.sigmoid(ff) * lc + jax.nn.sigmoid(ii) * jnp.tanh(gg)
        lh = jax.nn.sigmoid(oo) * jnp.tanh(lc)
        q = lh
        qt = jnp.tile(q, (1, _G))
        e8 = _hi(h * qt, efold_ref[...])
        e8 = e8 - jnp.max(e8)
        a8 = jnp.exp(e8)
        alpha_exp = _hi(a8 / jnp.sum(a8), aexp_ref[...])
        ro = jnp.sum(alpha_exp * h, axis=0, keepdims=True)
        readout = _hi(ro, rfold_ref[...])
        q_star = jnp.concatenate([q, readout], axis=-1)
    o_ref[...] = _bf(jnp.maximum(_bf(q_star, wm1_ref[...]) + bm1_ref[...],
                                 0.0), wm2_ref[...]) + bm2_ref[...]


def _make_sc_gather(n, h, e):
    ep = e // _NW
    ch = ep // _CHUNK
    mesh = plsc.VectorSubcoreMesh(core_axis_name="c", subcore_axis_name="s",
                                  num_cores=_NC, num_subcores=_NS)

    @functools.partial(
        pl.kernel,
        out_type=jax.ShapeDtypeStruct((e, h), jnp.float32),
        mesh=mesh,
        compiler_params=pltpu.CompilerParams(use_tc_tiling_on_sc=False),
        scratch_types=[
            pltpu.VMEM((ch, _CHUNK), jnp.int32),
            pltpu.VMEM((ep, h), jnp.float32),
            pltpu.SemaphoreType.DMA,
        ],
    )
    def sc_gather(idx_hbm, tab_hbm, out_hbm, idx_v, rows_v, sem):
        c = lax.axis_index("c")
        s = lax.axis_index("s")
        wid = s * _NC + c
        pltpu.sync_copy(idx_hbm.at[wid], idx_v)
        descs = [
            pltpu.async_copy(tab_hbm.at[idx_v.at[j]],
                             rows_v.at[pl.ds(j * _CHUNK, _CHUNK), :], sem)
            for j in range(ch)
        ]
        for d in descs:
            d.wait()
        pltpu.sync_copy(rows_v, out_hbm.at[pl.ds(wid * ep, ep), :])

    return sc_gather


def _make_sc_scatter(n_pad, h, e):
    ep = e // _NW
    ch = ep // _CHUNK
    rows = n_pad // _NS
    mesh = plsc.VectorSubcoreMesh(core_axis_name="c", subcore_axis_name="s",
                                  num_cores=_NC, num_subcores=_NS)

    @functools.partial(
        pl.kernel,
        out_type=jax.ShapeDtypeStruct((_NC, n_pad, h), jnp.float32),
        mesh=mesh,
        compiler_params=pltpu.CompilerParams(use_tc_tiling_on_sc=False),
        scratch_types=[
            pltpu.VMEM((ch, _CHUNK), jnp.int32),
            pltpu.VMEM((ep, h), jnp.float32),
            pltpu.VMEM((rows, h), jnp.float32),
            pltpu.VMEM_SHARED((n_pad, h), jnp.float32),
            pltpu.SemaphoreType.DMA,
        ],
    )
    def sc_scatter(idx_hbm, m_hbm, z_hbm, out_hbm, idx_v, m_v, z_v, acc_sh,
                   sem):
        c = lax.axis_index("c")
        s = lax.axis_index("s")
        wid = s * _NC + c
        pltpu.sync_copy(z_hbm, z_v)
        pltpu.sync_copy(z_v, acc_sh.at[pl.ds(s * rows, rows), :])
        pltpu.sync_copy(idx_hbm.at[wid], idx_v)
        pltpu.sync_copy(m_hbm.at[pl.ds(wid * ep, ep), :], m_v)
        plsc.subcore_barrier()
        for j in range(ch):
            pltpu.sync_copy(m_v.at[pl.ds(j * _CHUNK, _CHUNK), :],
                            acc_sh.at[idx_v.at[j]], add=True)
        plsc.subcore_barrier()
        pltpu.sync_copy(acc_sh.at[pl.ds(s * rows, rows), :],
                        out_hbm.at[c, pl.ds(s * rows, rows), :])

    return sc_scatter


_STEPS = 3
_S2S_STEPS = 6


def _bd(w):
    return jnp.kron(jnp.eye(_G, dtype=w.dtype), w)


def kernel(x, edge_index, edge_attr, W_proj, b_proj, W_e1, b_e1, W_e2, b_e2,
           b_conv, W_ih, W_hh, b_ih, b_hh, Wl_ih, Wl_hh, bl_ih, bl_hh,
           W_m1, b_m1, W_m2, b_m2):
    n, d_in = x.shape
    e, d_e = edge_attr.shape
    hd = W_proj.shape[0]
    eh = W_e1.shape[0]
    ep = e // _NW
    ch = ep // _CHUNK
    assert ep * _NW == e and ch * _CHUNK == ep
    assert n % _G == 0 and n % _NS == 0

    src3 = edge_index[0].reshape(_NW, ch, _CHUNK)
    dst3 = edge_index[1].reshape(_NW, ch, _CHUNK)

    ar_h = jnp.arange(hd)
    ar_hh = jnp.arange(hd * hd)
    r_mat = (ar_hh[None, :] // hd == ar_h[:, None]).astype(jnp.float32)
    s_mat = (ar_hh[:, None] % hd == ar_h[None, :]).astype(jnp.float32)
    zeros_rows = jnp.zeros((n // _NS, hd), jnp.float32)

    hrows = n // _G
    erows = e // _G

    h8 = pl.pallas_call(
        _proj_body,
        out_shape=jax.ShapeDtypeStruct((hrows, _G * hd), jnp.float32),
    )(x.reshape(hrows, _G * d_in), _bd(W_proj.T),
      jnp.tile(b_proj, _G).reshape(1, _G * hd))

    bf = 2000
    f8 = pl.pallas_call(
        _edge_f_body,
        grid=(erows // bf,),
        in_specs=[
            pl.BlockSpec((bf, _G * d_e), lambda i: (i, 0)),
            pl.BlockSpec((_G * d_e, _G * eh), lambda i: (0, 0)),
            pl.BlockSpec((1, _G * eh), lambda i: (0, 0)),
        ],
        out_specs=pl.BlockSpec((bf, _G * eh), lambda i: (i, 0)),
        out_shape=jax.ShapeDtypeStruct((erows, _G * eh), jnp.float32),
    )(edge_attr.reshape(erows, _G * d_e), _bd(W_e1.T),
      jnp.tile(b_e1, _G).reshape(1, _G * eh))

    sc_gather = _make_sc_gather(n, hd, e)
    sc_scatter = _make_sc_scatter(n, hd, e)

    bw = 400
    msg_call = pl.pallas_call(
        _msg_body,
        grid=(erows // bw,),
        in_specs=[
            pl.BlockSpec((bw, _G * hd), lambda i: (i, 0)),
            pl.BlockSpec((bw, _G * eh), lambda i: (i, 0)),
            pl.BlockSpec((_G * eh, _G * hd * hd), lambda i: (0, 0)),
            pl.BlockSpec((1, _G * hd * hd), lambda i: (0, 0)),
            pl.BlockSpec((_G * hd, _G * hd * hd), lambda i: (0, 0)),
            pl.BlockSpec((_G * hd * hd, _G * hd), lambda i: (0, 0)),
        ],
        out_specs=pl.BlockSpec((bw, _G * hd), lambda i: (i, 0)),
        out_shape=jax.ShapeDtypeStruct((erows, _G * hd), jnp.float32),
    )

    gru_call = pl.pallas_call(
        functools.partial(_gru_body, hrows),
        out_shape=jax.ShapeDtypeStruct((hrows, _G * hd), jnp.float32),
    )

    w2bd = _bd(W_e2.T)
    be2t = jnp.tile(b_e2, _G).reshape(1, _G * hd * hd)
    rbd = _bd(r_mat).astype(jnp.bfloat16)
    sbd = _bd(s_mat).astype(jnp.bfloat16)
    wih_t = W_ih.T
    whh_t = W_hh.T
    gru_w = [_bd(wih_t[:, g * hd:(g + 1) * hd]) for g in range(3)]
    gru_u = [_bd(whh_t[:, g * hd:(g + 1) * hd]) for g in range(3)]
    bi3 = jnp.stack([jnp.tile(b_ih[g * hd:(g + 1) * hd], _G)
                     for g in range(3)])
    bh3 = jnp.stack([jnp.tile(b_hh[g * hd:(g + 1) * hd], _G)
                     for g in range(3)])
    bconv_t = jnp.tile(b_conv, _G).reshape(1, _G * hd)

    for _ in range(_STEPS):
        h_tab = h8.reshape(n, hd)
        h_src = sc_gather(src3, h_tab)
        m8 = msg_call(h_src.reshape(erows, _G * hd), f8, w2bd, be2t,
                      rbd, sbd)
        parts = sc_scatter(dst3, m8.reshape(e, hd), zeros_rows)
        h8 = gru_call(parts.reshape(2 * hrows, _G * hd), h8, gru_w[0],
                      gru_w[1], gru_w[2], gru_u[0], gru_u[1], gru_u[2],
                      bi3, bh3, bconv_t)

    efold = jnp.kron(jnp.eye(_G, dtype=jnp.float32),
                     jnp.ones((hd, 1), jnp.float32))
    aexp = jnp.kron(jnp.eye(_G, dtype=jnp.float32),
                    jnp.ones((1, hd), jnp.float32))
    rfold = jnp.kron(jnp.ones((_G, 1), jnp.float32),
                     jnp.eye(hd, dtype=jnp.float32))

    out = pl.pallas_call(
        functools.partial(_s2s_body, _S2S_STEPS, hd),
        out_shape=jax.ShapeDtypeStruct((1, W_m2.shape[0]), jnp.float32),
    )(h8, efold, aexp, rfold, Wl_ih.T, Wl_hh.T, bl_ih.reshape(1, 4 * hd),
      bl_hh.reshape(1, 4 * hd), W_m1.T, b_m1.reshape(1, hd), W_m2.T,
      b_m2.reshape(1, W_m2.shape[0]))
    return out


# speedup vs baseline: 5.5076x; 1.9371x over previous
import functools

import jax
import jax.numpy as jnp
from jax import lax
from jax.experimental import pallas as pl
from jax.experimental.pallas import tpu as pltpu
from jax.experimental.pallas import tpu_sc as plsc

_NC, _NS = 2, 16
_NW = _NC * _NS
_CHUNK = 125
_G = 8


def _hi(a, b):
    return jnp.dot(a, b, precision=lax.Precision.HIGHEST,
                   preferred_element_type=jnp.float32)


def _bf(a, b):
    return jnp.dot(a, b, preferred_element_type=jnp.float32)


def _onehot_exact(a, p_ref):
    hi = a.astype(jnp.bfloat16)
    lo = (a - hi.astype(jnp.float32)).astype(jnp.bfloat16)
    return _bf(hi, p_ref[...]) + _bf(lo, p_ref[...])


def _proj_body(x_ref, w_ref, b_ref, o_ref):
    o_ref[...] = jnp.maximum(_bf(x_ref[...], w_ref[...]) + b_ref[...], 0.0)


def _edge_f_body(ea_ref, w_ref, b_ref, o_ref):
    o_ref[...] = jnp.maximum(_bf(ea_ref[...], w_ref[...]) + b_ref[...], 0.0)


def _msg_body(hd, hs_ref, f_ref, w2_ref, be2_ref, r_ref, o_ref):
    ew = _bf(f_ref[...], w2_ref[...]) + be2_ref[...]
    he = _onehot_exact(hs_ref[...], r_ref)
    z = he * ew
    acc = z[:, 0:_G * hd]
    for i in range(1, hd):
        acc = acc + z[:, i * _G * hd:(i + 1) * _G * hd]
    o_ref[...] = acc


def _gru_body(hrows, parts_ref, h_ref, wr_ref, wz_ref, wn_ref, ur_ref,
              uz_ref, un_ref, bi_ref, bh_ref, bconv_ref, o_ref):
    agg = parts_ref[:hrows, :] + parts_ref[hrows:2 * hrows, :] + bconv_ref[...]
    nf = jnp.maximum(agg, 0.0)
    h = h_ref[...]
    ir = _bf(nf, wr_ref[...]) + bi_ref[0:1, :]
    iz = _bf(nf, wz_ref[...]) + bi_ref[1:2, :]
    inn = _bf(nf, wn_ref[...]) + bi_ref[2:3, :]
    hr = _bf(h, ur_ref[...]) + bh_ref[0:1, :]
    hz = _bf(h, uz_ref[...]) + bh_ref[1:2, :]
    hn = _bf(h, un_ref[...]) + bh_ref[2:3, :]
    r = jax.nn.sigmoid(ir + hr)
    z = jax.nn.sigmoid(iz + hz)
    nn = jnp.tanh(inn + r * hn)
    o_ref[...] = (1.0 - z) * nn + z * h


def _s2s_body(steps, hd, h_ref, efold_ref, aexp_ref, rfold_ref, wlih_ref,
              wlhh_ref, blih_ref, blhh_ref, wm1_ref, bm1_ref, wm2_ref,
              bm2_ref, o_ref):
    h = h_ref[...]
    q_star = jnp.zeros((1, 2 * hd), jnp.float32)
    lh = jnp.zeros((1, hd), jnp.float32)
    lc = jnp.zeros((1, hd), jnp.float32)
    for _ in range(steps):
        gates = (_bf(q_star, wlih_ref[...]) + blih_ref[...]
                 + _bf(lh, wlhh_ref[...]) + blhh_ref[...])
        ii, ff, gg, oo = jnp.split(gates, 4, axis=-1)
        lc = jax.nn.sigmoid(ff) * lc + jax.nn.sigmoid(ii) * jnp.tanh(gg)
        lh = jax.nn.sigmoid(oo) * jnp.tanh(lc)
        q = lh
        qt = jnp.tile(q, (1, _G))
        e8 = _hi(h * qt, efold_ref[...])
        e8 = e8 - jnp.max(e8)
        a8 = jnp.exp(e8)
        alpha_exp = _hi(a8 / jnp.sum(a8), aexp_ref[...])
        ro = jnp.sum(alpha_exp * h, axis=0, keepdims=True)
        readout = _hi(ro, rfold_ref[...])
        q_star = jnp.concatenate([q, readout], axis=-1)
    o_ref[...] = _bf(jnp.maximum(_bf(q_star, wm1_ref[...]) + bm1_ref[...],
                                 0.0), wm2_ref[...]) + bm2_ref[...]


def _make_sc_gather(n, h, e):
    ep = e // _NW
    ch = ep // _CHUNK
    mesh = plsc.VectorSubcoreMesh(core_axis_name="c", subcore_axis_name="s",
                                  num_cores=_NC, num_subcores=_NS)

    @functools.partial(
        pl.kernel,
        out_type=jax.ShapeDtypeStruct((e, h), jnp.float32),
        mesh=mesh,
        compiler_params=pltpu.CompilerParams(use_tc_tiling_on_sc=False),
        scratch_types=[
            pltpu.VMEM((ch, _CHUNK), jnp.int32),
            pltpu.VMEM((ep, h), jnp.float32),
            pltpu.SemaphoreType.DMA,
        ],
    )
    def sc_gather(idx_hbm, tab_hbm, out_hbm, idx_v, rows_v, sem):
        c = lax.axis_index("c")
        s = lax.axis_index("s")
        wid = s * _NC + c
        pltpu.sync_copy(idx_hbm.at[wid], idx_v)
        descs = [
            pltpu.async_copy(tab_hbm.at[idx_v.at[j]],
                             rows_v.at[pl.ds(j * _CHUNK, _CHUNK), :], sem)
            for j in range(ch)
        ]
        for d in descs:
            d.wait()
        pltpu.sync_copy(rows_v, out_hbm.at[pl.ds(wid * ep, ep), :])

    return sc_gather


def _make_sc_scatter(n_pad, h, e):
    ep = e // _NW
    ch = ep // _CHUNK
    rows = n_pad // _NS
    mesh = plsc.VectorSubcoreMesh(core_axis_name="c", subcore_axis_name="s",
                                  num_cores=_NC, num_subcores=_NS)

    @functools.partial(
        pl.kernel,
        out_type=jax.ShapeDtypeStruct((_NC, n_pad, h), jnp.float32),
        mesh=mesh,
        compiler_params=pltpu.CompilerParams(use_tc_tiling_on_sc=False),
        scratch_types=[
            pltpu.VMEM((ch, _CHUNK), jnp.int32),
            pltpu.VMEM((ep, h), jnp.float32),
            pltpu.VMEM((rows, h), jnp.float32),
            pltpu.VMEM_SHARED((n_pad, h), jnp.float32),
            pltpu.SemaphoreType.DMA,
        ],
    )
    def sc_scatter(idx_hbm, m_hbm, z_hbm, out_hbm, idx_v, m_v, z_v, acc_sh,
                   sem):
        c = lax.axis_index("c")
        s = lax.axis_index("s")
        wid = s * _NC + c
        pltpu.sync_copy(z_hbm, z_v)
        pltpu.sync_copy(z_v, acc_sh.at[pl.ds(s * rows, rows), :])
        pltpu.sync_copy(idx_hbm.at[wid], idx_v)
        pltpu.sync_copy(m_hbm.at[pl.ds(wid * ep, ep), :], m_v)
        plsc.subcore_barrier()
        for j in range(ch):
            pltpu.sync_copy(m_v.at[pl.ds(j * _CHUNK, _CHUNK), :],
                            acc_sh.at[idx_v.at[j]], add=True)
        plsc.subcore_barrier()
        pltpu.sync_copy(acc_sh.at[pl.ds(s * rows, rows), :],
                        out_hbm.at[c, pl.ds(s * rows, rows), :])

    return sc_scatter


_STEPS = 3
_S2S_STEPS = 6


def _bd(w):
    return jnp.kron(jnp.eye(_G, dtype=w.dtype), w)


def kernel(x, edge_index, edge_attr, W_proj, b_proj, W_e1, b_e1, W_e2, b_e2,
           b_conv, W_ih, W_hh, b_ih, b_hh, Wl_ih, Wl_hh, bl_ih, bl_hh,
           W_m1, b_m1, W_m2, b_m2):
    n, d_in = x.shape
    e, d_e = edge_attr.shape
    hd = W_proj.shape[0]
    eh = W_e1.shape[0]
    ep = e // _NW
    ch = ep // _CHUNK
    assert ep * _NW == e and ch * _CHUNK == ep
    assert n % _G == 0 and n % _NS == 0

    src3 = edge_index[0].reshape(_NW, ch, _CHUNK)
    dst3 = edge_index[1].reshape(_NW, ch, _CHUNK)

    ar_h = jnp.arange(hd)
    ar_hh = jnp.arange(hd * hd)
    r_mat = (ar_hh[None, :] // hd == ar_h[:, None]).astype(jnp.float32)
    s_mat = (ar_hh[:, None] % hd == ar_h[None, :]).astype(jnp.float32)
    zeros_rows = jnp.zeros((n // _NS, hd), jnp.float32)

    hrows = n // _G
    erows = e // _G

    h8 = pl.pallas_call(
        _proj_body,
        out_shape=jax.ShapeDtypeStruct((hrows, _G * hd), jnp.float32),
    )(x.reshape(hrows, _G * d_in), _bd(W_proj.T),
      jnp.tile(b_proj, _G).reshape(1, _G * hd))

    bf = 2000
    f8 = pl.pallas_call(
        _edge_f_body,
        grid=(erows // bf,),
        in_specs=[
            pl.BlockSpec((bf, _G * d_e), lambda i: (i, 0)),
            pl.BlockSpec((_G * d_e, _G * eh), lambda i: (0, 0)),
            pl.BlockSpec((1, _G * eh), lambda i: (0, 0)),
        ],
        out_specs=pl.BlockSpec((bf, _G * eh), lambda i: (i, 0)),
        out_shape=jax.ShapeDtypeStruct((erows, _G * eh), jnp.float32),
    )(edge_attr.reshape(erows, _G * d_e), _bd(W_e1.T),
      jnp.tile(b_e1, _G).reshape(1, _G * eh))

    sc_gather = _make_sc_gather(n, hd, e)
    sc_scatter = _make_sc_scatter(n, hd, e)

    bw = 400
    msg_call = pl.pallas_call(
        functools.partial(_msg_body, hd),
        grid=(erows // bw,),
        in_specs=[
            pl.BlockSpec((bw, _G * hd), lambda i: (i, 0)),
            pl.BlockSpec((bw, _G * eh), lambda i: (i, 0)),
            pl.BlockSpec((_G * eh, _G * hd * hd), lambda i: (0, 0)),
            pl.BlockSpec((1, _G * hd * hd), lambda i: (0, 0)),
            pl.BlockSpec((_G * hd, _G * hd * hd), lambda i: (0, 0)),
        ],
        out_specs=pl.BlockSpec((bw, _G * hd), lambda i: (i, 0)),
        out_shape=jax.ShapeDtypeStruct((erows, _G * hd), jnp.float32),
    )

    gru_call = pl.pallas_call(
        functools.partial(_gru_body, hrows),
        out_shape=jax.ShapeDtypeStruct((hrows, _G * hd), jnp.float32),
    )

    jj = jnp.arange(_G * hd * hd)
    i_of = jj // (_G * hd)
    el_of = (jj % (_G * hd)) // hd
    o_of = jj % hd
    col = el_of * (hd * hd) + i_of * hd + o_of
    w2bd = _bd(W_e2.T)[:, col]
    be2t = jnp.tile(b_e2, _G).reshape(1, _G * hd * hd)[:, col]
    src_lane = el_of * hd + i_of
    rbd = (jnp.arange(_G * hd)[:, None] == src_lane[None, :]).astype(
        jnp.bfloat16)
    wih_t = W_ih.T
    whh_t = W_hh.T
    gru_w = [_bd(wih_t[:, g * hd:(g + 1) * hd]) for g in range(3)]
    gru_u = [_bd(whh_t[:, g * hd:(g + 1) * hd]) for g in range(3)]
    bi3 = jnp.stack([jnp.tile(b_ih[g * hd:(g + 1) * hd], _G)
                     for g in range(3)])
    bh3 = jnp.stack([jnp.tile(b_hh[g * hd:(g + 1) * hd], _G)
                     for g in range(3)])
    bconv_t = jnp.tile(b_conv, _G).reshape(1, _G * hd)

    for _ in range(_STEPS):
        h_tab = h8.reshape(n, hd)
        h_src = sc_gather(src3, h_tab)
        m8 = msg_call(h_src.reshape(erows, _G * hd), f8, w2bd, be2t, rbd)
        parts = sc_scatter(dst3, m8.reshape(e, hd), zeros_rows)
        h8 = gru_call(parts.reshape(2 * hrows, _G * hd), h8, gru_w[0],
                      gru_w[1], gru_w[2], gru_u[0], gru_u[1], gru_u[2],
                      bi3, bh3, bconv_t)

    efold = jnp.kron(jnp.eye(_G, dtype=jnp.float32),
                     jnp.ones((hd, 1), jnp.float32))
    aexp = jnp.kron(jnp.eye(_G, dtype=jnp.float32),
                    jnp.ones((1, hd), jnp.float32))
    rfold = jnp.kron(jnp.ones((_G, 1), jnp.float32),
                     jnp.eye(hd, dtype=jnp.float32))

    out = pl.pallas_call(
        functools.partial(_s2s_body, _S2S_STEPS, hd),
        out_shape=jax.ShapeDtypeStruct((1, W_m2.shape[0]), jnp.float32),
    )(h8, efold, aexp, rfold, Wl_ih.T, Wl_hh.T, bl_ih.reshape(1, 4 * hd),
      bl_hh.reshape(1, 4 * hd), W_m1.T, b_m1.reshape(1, hd), W_m2.T,
      b_m2.reshape(1, W_m2.shape[0]))
    return out


# speedup vs baseline: 6.2353x; 1.1321x over previous
import functools

import jax
import jax.numpy as jnp
from jax import lax
from jax.experimental import pallas as pl
from jax.experimental.pallas import tpu as pltpu
from jax.experimental.pallas import tpu_sc as plsc

_NC, _NS = 2, 16
_NW = _NC * _NS
_CHUNK = 125
_G = 8


def _hi(a, b):
    return jnp.dot(a, b, precision=lax.Precision.HIGHEST,
                   preferred_element_type=jnp.float32)


def _bf(a, b):
    return jnp.dot(a, b, preferred_element_type=jnp.float32)


def _onehot_exact(a, p_ref):
    hi = a.astype(jnp.bfloat16)
    lo = (a - hi.astype(jnp.float32)).astype(jnp.bfloat16)
    return _bf(hi, p_ref[...]) + _bf(lo, p_ref[...])


def _proj_body(x_ref, w_ref, b_ref, o_ref):
    o_ref[...] = jnp.maximum(_bf(x_ref[...], w_ref[...]) + b_ref[...], 0.0)


def _edge_f_body(ea_ref, w_ref, b_ref, o_ref):
    o_ref[...] = jnp.maximum(_bf(ea_ref[...], w_ref[...]) + b_ref[...],
                             0.0).astype(jnp.bfloat16)


def _msg_body(hd, hs_ref, f_ref, w2_ref, be2_ref, r2_ref, o_ref):
    ew = _bf(f_ref[...], w2_ref[...]) + be2_ref[...]
    hs = hs_ref[...]
    hi = hs.astype(jnp.bfloat16)
    lo = (hs - hi.astype(jnp.float32)).astype(jnp.bfloat16)
    he = _bf(jnp.concatenate([hi, lo], axis=1), r2_ref[...])
    z = he * ew
    acc = z[:, 0:_G * hd]
    for i in range(1, hd):
        acc = acc + z[:, i * _G * hd:(i + 1) * _G * hd]
    o_ref[...] = acc


def _gru_body(hrows, parts_ref, h_ref, wr_ref, wz_ref, wn_ref, ur_ref,
              uz_ref, un_ref, bi_ref, bh_ref, bconv_ref, o_ref):
    agg = parts_ref[:hrows, :] + parts_ref[hrows:2 * hrows, :] + bconv_ref[...]
    nf = jnp.maximum(agg, 0.0)
    h = h_ref[...]
    ir = _bf(nf, wr_ref[...]) + bi_ref[0:1, :]
    iz = _bf(nf, wz_ref[...]) + bi_ref[1:2, :]
    inn = _bf(nf, wn_ref[...]) + bi_ref[2:3, :]
    hr = _bf(h, ur_ref[...]) + bh_ref[0:1, :]
    hz = _bf(h, uz_ref[...]) + bh_ref[1:2, :]
    hn = _bf(h, un_ref[...]) + bh_ref[2:3, :]
    r = jax.nn.sigmoid(ir + hr)
    z = jax.nn.sigmoid(iz + hz)
    nn = jnp.tanh(inn + r * hn)
    o_ref[...] = (1.0 - z) * nn + z * h


def _s2s_body(steps, hd, h_ref, efold_ref, aexp_ref, rfold_ref, wlih_ref,
              wlhh_ref, blih_ref, blhh_ref, wm1_ref, bm1_ref, wm2_ref,
              bm2_ref, o_ref):
    h = h_ref[...]
    q_star = jnp.zeros((1, 2 * hd), jnp.float32)
    lh = jnp.zeros((1, hd), jnp.float32)
    lc = jnp.zeros((1, hd), jnp.float32)
    for _ in range(steps):
        gates = (_bf(q_star, wlih_ref[...]) + blih_ref[...]
                 + _bf(lh, wlhh_ref[...]) + blhh_ref[...])
        ii, ff, gg, oo = jnp.split(gates, 4, axis=-1)
        lc = jax.nn.sigmoid(ff) * lc + jax.nn.sigmoid(ii) * jnp.tanh(gg)
        lh = jax.nn.sigmoid(oo) * jnp.tanh(lc)
        q = lh
        qt = jnp.tile(q, (1, _G))
        e8 = _hi(h * qt, efold_ref[...])
        e8 = e8 - jnp.max(e8)
        a8 = jnp.exp(e8)
        alpha_exp = _hi(a8 / jnp.sum(a8), aexp_ref[...])
        ro = jnp.sum(alpha_exp * h, axis=0, keepdims=True)
        readout = _hi(ro, rfold_ref[...])
        q_star = jnp.concatenate([q, readout], axis=-1)
    o_ref[...] = _bf(jnp.maximum(_bf(q_star, wm1_ref[...]) + bm1_ref[...],
                                 0.0), wm2_ref[...]) + bm2_ref[...]


def _make_sc_gather(n, h, e):
    ep = e // _NW
    ch = ep // _CHUNK
    mesh = plsc.VectorSubcoreMesh(core_axis_name="c", subcore_axis_name="s",
                                  num_cores=_NC, num_subcores=_NS)

    @functools.partial(
        pl.kernel,
        out_type=jax.ShapeDtypeStruct((e, h), jnp.float32),
        mesh=mesh,
        compiler_params=pltpu.CompilerParams(use_tc_tiling_on_sc=False),
        scratch_types=[
            pltpu.VMEM((ch, _CHUNK), jnp.int32),
            pltpu.VMEM((ep, h), jnp.float32),
            pltpu.SemaphoreType.DMA,
        ],
    )
    def sc_gather(idx_hbm, tab_hbm, out_hbm, idx_v, rows_v, sem):
        c = lax.axis_index("c")
        s = lax.axis_index("s")
        wid = s * _NC + c
        pltpu.sync_copy(idx_hbm.at[wid], idx_v)
        descs = [
            pltpu.async_copy(tab_hbm.at[idx_v.at[j]],
                             rows_v.at[pl.ds(j * _CHUNK, _CHUNK), :], sem)
            for j in range(ch)
        ]
        for d in descs:
            d.wait()
        pltpu.sync_copy(rows_v, out_hbm.at[pl.ds(wid * ep, ep), :])

    return sc_gather


def _make_sc_scatter(n_pad, h, e):
    ep = e // _NW
    ch = ep // _CHUNK
    rows = n_pad // _NS
    mesh = plsc.VectorSubcoreMesh(core_axis_name="c", subcore_axis_name="s",
                                  num_cores=_NC, num_subcores=_NS)

    @functools.partial(
        pl.kernel,
        out_type=jax.ShapeDtypeStruct((_NC, n_pad, h), jnp.float32),
        mesh=mesh,
        compiler_params=pltpu.CompilerParams(use_tc_tiling_on_sc=False),
        scratch_types=[
            pltpu.VMEM((ch, _CHUNK), jnp.int32),
            pltpu.VMEM((ep, h), jnp.float32),
            pltpu.VMEM((rows, h), jnp.float32),
            pltpu.VMEM_SHARED((n_pad, h), jnp.float32),
            pltpu.SemaphoreType.DMA,
        ],
    )
    def sc_scatter(idx_hbm, m_hbm, z_hbm, out_hbm, idx_v, m_v, z_v, acc_sh,
                   sem):
        c = lax.axis_index("c")
        s = lax.axis_index("s")
        wid = s * _NC + c
        pltpu.sync_copy(z_hbm, z_v)
        pltpu.sync_copy(z_v, acc_sh.at[pl.ds(s * rows, rows), :])
        pltpu.sync_copy(idx_hbm.at[wid], idx_v)
        pltpu.sync_copy(m_hbm.at[pl.ds(wid * ep, ep), :], m_v)
        plsc.subcore_barrier()
        for j in range(ch):
            pltpu.sync_copy(m_v.at[pl.ds(j * _CHUNK, _CHUNK), :],
                            acc_sh.at[idx_v.at[j]], add=True)
        plsc.subcore_barrier()
        pltpu.sync_copy(acc_sh.at[pl.ds(s * rows, rows), :],
                        out_hbm.at[c, pl.ds(s * rows, rows), :])

    return sc_scatter


_STEPS = 3
_S2S_STEPS = 6


def _bd(w):
    return jnp.kron(jnp.eye(_G, dtype=w.dtype), w)


def kernel(x, edge_index, edge_attr, W_proj, b_proj, W_e1, b_e1, W_e2, b_e2,
           b_conv, W_ih, W_hh, b_ih, b_hh, Wl_ih, Wl_hh, bl_ih, bl_hh,
           W_m1, b_m1, W_m2, b_m2):
    n, d_in = x.shape
    e, d_e = edge_attr.shape
    hd = W_proj.shape[0]
    eh = W_e1.shape[0]
    ep = e // _NW
    ch = ep // _CHUNK
    assert ep * _NW == e and ch * _CHUNK == ep
    assert n % _G == 0 and n % _NS == 0

    src3 = edge_index[0].reshape(_NW, ch, _CHUNK)
    dst3 = edge_index[1].reshape(_NW, ch, _CHUNK)

    ar_h = jnp.arange(hd)
    ar_hh = jnp.arange(hd * hd)
    r_mat = (ar_hh[None, :] // hd == ar_h[:, None]).astype(jnp.float32)
    s_mat = (ar_hh[:, None] % hd == ar_h[None, :]).astype(jnp.float32)
    zeros_rows = jnp.zeros((n // _NS, hd), jnp.float32)

    hrows = n // _G
    erows = e // _G

    h8 = pl.pallas_call(
        _proj_body,
        out_shape=jax.ShapeDtypeStruct((hrows, _G * hd), jnp.float32),
    )(x.reshape(hrows, _G * d_in), _bd(W_proj.T),
      jnp.tile(b_proj, _G).reshape(1, _G * hd))

    bf = 2000
    f8 = pl.pallas_call(
        _edge_f_body,
        grid=(erows // bf,),
        in_specs=[
            pl.BlockSpec((bf, _G * d_e), lambda i: (i, 0)),
            pl.BlockSpec((_G * d_e, _G * eh), lambda i: (0, 0)),
            pl.BlockSpec((1, _G * eh), lambda i: (0, 0)),
        ],
        out_specs=pl.BlockSpec((bf, _G * eh), lambda i: (i, 0)),
        out_shape=jax.ShapeDtypeStruct((erows, _G * eh), jnp.bfloat16),
    )(edge_attr.reshape(erows, _G * d_e), _bd(W_e1.T),
      jnp.tile(b_e1, _G).reshape(1, _G * eh))

    sc_gather = _make_sc_gather(n, hd, e)
    sc_scatter = _make_sc_scatter(n, hd, e)

    bw = 400
    msg_call = pl.pallas_call(
        functools.partial(_msg_body, hd),
        grid=(erows // bw,),
        in_specs=[
            pl.BlockSpec((bw, _G * hd), lambda i: (i, 0)),
            pl.BlockSpec((bw, _G * eh), lambda i: (i, 0)),
            pl.BlockSpec((_G * eh, _G * hd * hd), lambda i: (0, 0)),
            pl.BlockSpec((1, _G * hd * hd), lambda i: (0, 0)),
            pl.BlockSpec((2 * _G * hd, _G * hd * hd), lambda i: (0, 0)),
        ],
        out_specs=pl.BlockSpec((bw, _G * hd), lambda i: (i, 0)),
        out_shape=jax.ShapeDtypeStruct((erows, _G * hd), jnp.float32),
    )

    gru_call = pl.pallas_call(
        functools.partial(_gru_body, hrows),
        out_shape=jax.ShapeDtypeStruct((hrows, _G * hd), jnp.float32),
    )

    jj = jnp.arange(_G * hd * hd)
    i_of = jj // (_G * hd)
    el_of = (jj % (_G * hd)) // hd
    o_of = jj % hd
    col = el_of * (hd * hd) + i_of * hd + o_of
    w2bd = _bd(W_e2.T)[:, col]
    be2t = jnp.tile(b_e2, _G).reshape(1, _G * hd * hd)[:, col]
    src_lane = el_of * hd + i_of
    rbd = (jnp.arange(_G * hd)[:, None] == src_lane[None, :]).astype(
        jnp.bfloat16)
    rbd2 = jnp.concatenate([rbd, rbd], axis=0)
    wih_t = W_ih.T
    whh_t = W_hh.T
    gru_w = [_bd(wih_t[:, g * hd:(g + 1) * hd]) for g in range(3)]
    gru_u = [_bd(whh_t[:, g * hd:(g + 1) * hd]) for g in range(3)]
    bi3 = jnp.stack([jnp.tile(b_ih[g * hd:(g + 1) * hd], _G)
                     for g in range(3)])
    bh3 = jnp.stack([jnp.tile(b_hh[g * hd:(g + 1) * hd], _G)
                     for g in range(3)])
    bconv_t = jnp.tile(b_conv, _G).reshape(1, _G * hd)

    for _ in range(_STEPS):
        h_tab = h8.reshape(n, hd)
        h_src = sc_gather(src3, h_tab)
        m8 = msg_call(h_src.reshape(erows, _G * hd), f8, w2bd, be2t, rbd2)
        parts = sc_scatter(dst3, m8.reshape(e, hd), zeros_rows)
        h8 = gru_call(parts.reshape(2 * hrows, _G * hd), h8, gru_w[0],
                      gru_w[1], gru_w[2], gru_u[0], gru_u[1], gru_u[2],
                      bi3, bh3, bconv_t)

    efold = jnp.kron(jnp.eye(_G, dtype=jnp.float32),
                     jnp.ones((hd, 1), jnp.float32))
    aexp = jnp.kron(jnp.eye(_G, dtype=jnp.float32),
                    jnp.ones((1, hd), jnp.float32))
    rfold = jnp.kron(jnp.ones((_G, 1), jnp.float32),
                     jnp.eye(hd, dtype=jnp.float32))

    out = pl.pallas_call(
        functools.partial(_s2s_body, _S2S_STEPS, hd),
        out_shape=jax.ShapeDtypeStruct((1, W_m2.shape[0]), jnp.float32),
    )(h8, efold, aexp, rfold, Wl_ih.T, Wl_hh.T, bl_ih.reshape(1, 4 * hd),
      bl_hh.reshape(1, 4 * hd), W_m1.T, b_m1.reshape(1, hd), W_m2.T,
      b_m2.reshape(1, W_m2.shape[0]))
    return out
